# baseline (speedup 1.0000x reference)
import functools

import jax
import jax.numpy as jnp
from jax import lax
from jax.experimental import pallas as pl
from jax.experimental.pallas import tpu as pltpu
from jax.experimental.pallas import tpu_sc as plsc

N_NODES = 10000
D_FEAT = 128
D_HID = 128
N_CLASSES = 40
N_EDGES = 320000
MIN_NORM = 1e-15
MAXNORM = 1.0 - 1e-5

ROWS_PER_BLOCK = 2000
N_BLOCKS = N_NODES // ROWS_PER_BLOCK

EDGE_CHUNK = 40
N_SC_CORES = 2
N_SC_SUBCORES = 16
EDGES_PER_TILE = N_EDGES // (N_SC_CORES * N_SC_SUBCORES)
CHUNKS_PER_TILE = EDGES_PER_TILE // EDGE_CHUNK
ROWS_MAIN = 624
ROWS_REM = N_NODES - N_SC_SUBCORES * ROWS_MAIN


def _artanh(z):
    z = jnp.clip(z, -1.0 + 1e-7, 1.0 - 1e-7)
    return 0.5 * jnp.log((1.0 + z) / (1.0 - z))


def _tanh(z):
    return jnp.tanh(jnp.clip(z, -15.0, 15.0))


def _norm(v):
    return jnp.sqrt(jnp.sum(v * v, axis=-1, keepdims=True))


def _proj(v):
    n = jnp.maximum(_norm(v), MIN_NORM)
    return jnp.where(n > MAXNORM, v / n * MAXNORM, v)


def _expmap0(u):
    n = jnp.maximum(_norm(u), MIN_NORM)
    return _tanh(n) * u / n


def _logmap0(p):
    n = jnp.maximum(_norm(p), MIN_NORM)
    return p / n * _artanh(n)


def _mobius_add(a, b):
    a2 = jnp.sum(a * a, axis=-1, keepdims=True)
    b2 = jnp.sum(b * b, axis=-1, keepdims=True)
    ab = jnp.sum(a * b, axis=-1, keepdims=True)
    num = (1.0 + 2.0 * ab + b2) * a + (1.0 - a2) * b
    den = 1.0 + 2.0 * ab + a2 * b2
    return num / jnp.maximum(den, MIN_NORM)


def _hyp_linear(h, W, brow):
    hn = jnp.maximum(_norm(h), MIN_NORM)
    mx = lax.dot_general(h, W, (((1,), (1,)), ((), ())),
                         preferred_element_type=jnp.float32)
    mxn = jnp.maximum(_norm(mx), MIN_NORM)
    mv = _tanh(mxn / hn * _artanh(hn)) * mx / mxn
    mv = _proj(mv)
    bias = _proj(_expmap0(brow))
    return _proj(_mobius_add(mv, bias))


def _stage1_body(x_ref, w_ref, b_ref, o_ref):
    x = x_ref[...]
    xh = _proj(_expmap0(x))
    res = _hyp_linear(xh, w_ref[...], b_ref[...])
    o_ref[...] = _logmap0(res)


def _stage2_body(p0_ref, p1_ref, w_ref, b_ref, o_ref):
    agg = p0_ref[...] + p1_ref[...]
    out = _proj(_expmap0(agg))
    out = _proj(_expmap0(jax.nn.relu(_logmap0(out))))
    res = _hyp_linear(out, w_ref[...], b_ref[...])
    o_ref[...] = _logmap0(res)


def _stage3_body(p0_ref, p1_ref, wd_ref, bd_ref, o_ref):
    agg = p0_ref[...] + p1_ref[...]
    h = _proj(_expmap0(agg))
    ht = _logmap0(h)
    logits = lax.dot_general(ht, wd_ref[...], (((1,), (1,)), ((), ())),
                             preferred_element_type=jnp.float32)
    logits = logits + bd_ref[...]
    m = jnp.max(logits, axis=-1, keepdims=True)
    sh = logits - m
    lse = jnp.log(jnp.sum(jnp.exp(sh), axis=-1, keepdims=True))
    o_ref[...] = sh - lse


def _row_spec():
    return pl.BlockSpec((ROWS_PER_BLOCK, D_HID), lambda i: (i, 0))


def _part_spec(k):
    return pl.BlockSpec((1, ROWS_PER_BLOCK, D_HID), lambda i, k=k: (k, i, 0))


def _full_spec(r, c):
    return pl.BlockSpec((r, c), lambda i: (0, 0))


def _tc_stage1(x, W1, b1):
    return pl.pallas_call(
        _stage1_body,
        grid=(N_BLOCKS,),
        in_specs=[_row_spec(), _full_spec(D_HID, D_FEAT), _full_spec(1, D_HID)],
        out_specs=_row_spec(),
        out_shape=jax.ShapeDtypeStruct((N_NODES, D_HID), jnp.float32),
    )(x, W1, b1.reshape(1, D_HID))


def _tc_stage2(p0, p1, W2, b2):
    return pl.pallas_call(
        _stage2_body,
        grid=(N_BLOCKS,),
        in_specs=[_row_spec(), _row_spec(), _full_spec(D_HID, D_HID),
                  _full_spec(1, D_HID)],
        out_specs=_row_spec(),
        out_shape=jax.ShapeDtypeStruct((N_NODES, D_HID), jnp.float32),
    )(p0, p1, W2, b2.reshape(1, D_HID))


def _tc_stage3(p0, p1, Wd, bd):
    return pl.pallas_call(
        _stage3_body,
        grid=(N_BLOCKS,),
        in_specs=[_row_spec(), _row_spec(), _full_spec(N_CLASSES, D_HID),
                  _full_spec(1, N_CLASSES)],
        out_specs=pl.BlockSpec((ROWS_PER_BLOCK, N_CLASSES), lambda i: (i, 0)),
        out_shape=jax.ShapeDtypeStruct((N_NODES, N_CLASSES), jnp.float32),
    )(p0, p1, Wd, bd.reshape(1, N_CLASSES))


RING = 4


def _sc_agg_kernel(xt_hbm, src_hbm, dst_hbm, w16_hbm, zero_hbm, out_hbm,
                   *sc):
    cid = lax.axis_index("c")
    sid = lax.axis_index("s")
    wid = cid * N_SC_SUBCORES + sid

    sidx = sc[0:4]
    didx = sc[4:6]
    w16s = sc[6:8]
    rows = sc[8:12]
    sbuf = sc[12:14]
    acc = sc[14]
    gsems = sc[15:19]
    wsems = sc[19:21]
    isems = sc[21:25]
    idsems = sc[25:27]
    ssems = sc[27:29]
    LAST = CHUNKS_PER_TILE - 1

    pltpu.sync_copy(zero_hbm, acc.at[pl.ds(sid * ROWS_MAIN, ROWS_MAIN)])

    @pl.when(sid == N_SC_SUBCORES - 1)
    def _zero_rem():
        pltpu.sync_copy(zero_hbm.at[pl.ds(0, ROWS_REM)],
                        acc.at[pl.ds(N_SC_SUBCORES * ROWS_MAIN, ROWS_REM)])

    plsc.subcore_barrier()

    def _fire_sidx(r, c):
        pltpu.async_copy(src_hbm.at[wid, c], sidx[r], isems[r])

    def _wait_sidx(r):
        pltpu.make_async_copy(src_hbm.at[wid, LAST], sidx[r],
                              isems[r]).wait()

    def _fire_gather(r, c):
        pltpu.async_copy(xt_hbm.at[sidx[r]], rows[r], gsems[r])

    def _wait_gather(r):
        pltpu.make_async_copy(xt_hbm.at[sidx[r]], rows[r], gsems[r]).wait()

    def _fire_w16(b, c):
        pltpu.async_copy(w16_hbm.at[wid, c], w16s[b], wsems[b])

    def _wait_w16(b):
        pltpu.make_async_copy(w16_hbm.at[wid, LAST], w16s[b],
                              wsems[b]).wait()

    def _wait_scatter(b):
        pltpu.make_async_copy(sbuf[b], acc.at[didx[b]], ssems[b]).wait()

    def _scale(r, b):
        for e in range(EDGE_CHUNK):
            wv = w16s[b][e, :]
            for cb in range(8):
                sl = pl.ds(cb * 16, 16)
                sbuf[b][e, sl] = rows[r][e, sl] * wv

    def _chunk_body(c, r, b, wait_scatter):
        _wait_gather(r)
        _wait_w16(b)
        r3 = (r + 3) % RING
        _wait_sidx(r3)
        _fire_gather(r3, jnp.minimum(c + 3, LAST))
        if wait_scatter is None:
            _wait_scatter(b)
        else:
            @pl.when(wait_scatter)
            def _wait_prev():
                _wait_scatter(b)
        _fire_sidx(r, jnp.minimum(c + 4, LAST))
        pltpu.async_copy(dst_hbm.at[wid, c], didx[b], idsems[b])
        _scale(r, b)
        _fire_w16(b, jnp.minimum(c + 2, LAST))
        pltpu.make_async_copy(dst_hbm.at[wid, c], didx[b],
                              idsems[b]).wait()
        pltpu.async_copy(sbuf[b], acc.at[didx[b]], ssems[b], add=True)

    for k in range(RING):
        _fire_sidx(k, k)
    for k in range(RING - 1):
        _wait_sidx(k)
        _fire_gather(k, k)
    _fire_w16(0, 0)
    _fire_w16(1, 1)

    def group(i, carry):
        for u in range(RING):
            c = RING * i + u
            guard = (i > 0) if u < 2 else None
            _chunk_body(c, u, u % 2, guard)
        return carry

    n_groups = (CHUNKS_PER_TILE - 2) // RING
    lax.fori_loop(0, n_groups, group, 0)
    g0 = RING * n_groups
    _chunk_body(g0, 0, 0, None)
    _chunk_body(g0 + 1, 1, 1, None)

    _wait_sidx(1)
    _wait_gather(0)
    _wait_gather(2)
    _wait_gather(3)
    _wait_w16(0)
    _wait_w16(1)
    _wait_scatter(0)
    _wait_scatter(1)

    plsc.subcore_barrier()
    r0 = sid * ROWS_MAIN
    pltpu.sync_copy(acc.at[pl.ds(r0, ROWS_MAIN)],
                    out_hbm.at[cid, pl.ds(r0, ROWS_MAIN)])

    @pl.when(sid == N_SC_SUBCORES - 1)
    def _write_rem():
        rr = N_SC_SUBCORES * ROWS_MAIN
        pltpu.sync_copy(acc.at[pl.ds(rr, ROWS_REM)],
                        out_hbm.at[cid, pl.ds(rr, ROWS_REM)])


def _sc_agg(xt, src, dst, w16, zeros_tile):
    mesh = plsc.VectorSubcoreMesh(core_axis_name="c", subcore_axis_name="s")
    f = functools.partial(
        pl.kernel,
        out_type=jax.ShapeDtypeStruct((N_SC_CORES, N_NODES, D_HID),
                                      jnp.float32),
        mesh=mesh,
        scratch_types=(
            [pltpu.VMEM((EDGE_CHUNK,), jnp.int32)] * 6
            + [pltpu.VMEM((EDGE_CHUNK, 16), jnp.float32)] * 2
            + [pltpu.VMEM((EDGE_CHUNK, D_HID), jnp.float32)] * 6
            + [pltpu.VMEM_SHARED((N_NODES, D_HID), jnp.float32)]
            + [pltpu.SemaphoreType.DMA] * 14
        ),
    )(_sc_agg_kernel)
    return f(xt, src, dst, w16, zeros_tile)


def kernel(x, edge_index, edge_weight, W1, b1, W2, b2, Wd, bd):
    n_tiles = N_SC_CORES * N_SC_SUBCORES
    src = edge_index[0].astype(jnp.int32).reshape(
        n_tiles, CHUNKS_PER_TILE, EDGE_CHUNK)
    dst = edge_index[1].astype(jnp.int32).reshape(
        n_tiles, CHUNKS_PER_TILE, EDGE_CHUNK)
    w16 = jnp.broadcast_to(edge_weight.astype(jnp.float32)[:, None],
                           (N_EDGES, 16)).reshape(
        n_tiles, CHUNKS_PER_TILE, EDGE_CHUNK, 16)
    zeros_tile = jnp.zeros((ROWS_MAIN, D_HID), jnp.float32)

    xt1 = _tc_stage1(x, W1, b1)
    parts1 = _sc_agg(xt1, src, dst, w16, zeros_tile)
    xt2 = _tc_stage2(parts1[0], parts1[1], W2, b2)
    parts2 = _sc_agg(xt2, src, dst, w16, zeros_tile)
    return _tc_stage3(parts2[0], parts2[1], Wd, bd)


# speedup vs baseline: 1.0420x; 1.0420x over previous
import functools

import jax
import jax.numpy as jnp
from jax import lax
from jax.experimental import pallas as pl
from jax.experimental.pallas import tpu as pltpu
from jax.experimental.pallas import tpu_sc as plsc

N_NODES = 10000
D_FEAT = 128
D_HID = 128
N_CLASSES = 40
N_EDGES = 320000
MIN_NORM = 1e-15
MAXNORM = 1.0 - 1e-5

ROWS_PER_BLOCK = 1000
N_BLOCKS = N_NODES // ROWS_PER_BLOCK

EDGE_CHUNK = 40
N_SC_CORES = 2
N_SC_SUBCORES = 16
EDGES_PER_TILE = N_EDGES // (N_SC_CORES * N_SC_SUBCORES)
CHUNKS_PER_TILE = EDGES_PER_TILE // EDGE_CHUNK
ROWS_MAIN = 624
ROWS_REM = N_NODES - N_SC_SUBCORES * ROWS_MAIN


def _artanh(z):
    z = jnp.clip(z, -1.0 + 1e-7, 1.0 - 1e-7)
    return 0.5 * jnp.log((1.0 + z) / (1.0 - z))


def _tanh(z):
    return jnp.tanh(jnp.clip(z, -15.0, 15.0))


def _norm(v):
    return jnp.sqrt(jnp.sum(v * v, axis=-1, keepdims=True))


def _proj(v):
    n = jnp.maximum(_norm(v), MIN_NORM)
    return jnp.where(n > MAXNORM, v / n * MAXNORM, v)


def _expmap0(u):
    n = jnp.maximum(_norm(u), MIN_NORM)
    return _tanh(n) * u / n


def _logmap0(p):
    n = jnp.maximum(_norm(p), MIN_NORM)
    return p / n * _artanh(n)


def _mobius_add(a, b):
    a2 = jnp.sum(a * a, axis=-1, keepdims=True)
    b2 = jnp.sum(b * b, axis=-1, keepdims=True)
    ab = jnp.sum(a * b, axis=-1, keepdims=True)
    num = (1.0 + 2.0 * ab + b2) * a + (1.0 - a2) * b
    den = 1.0 + 2.0 * ab + a2 * b2
    return num / jnp.maximum(den, MIN_NORM)


def _hyp_linear(h, W, brow):
    hn = jnp.maximum(_norm(h), MIN_NORM)
    mx = lax.dot_general(h, W, (((1,), (1,)), ((), ())),
                         preferred_element_type=jnp.float32)
    mxn = jnp.maximum(_norm(mx), MIN_NORM)
    mv = _tanh(mxn / hn * _artanh(hn)) * mx / mxn
    mv = _proj(mv)
    bias = _proj(_expmap0(brow))
    return _proj(_mobius_add(mv, bias))


def _stage1_body(x_ref, w_ref, b_ref, o_ref):
    x = x_ref[...]
    xh = _proj(_expmap0(x))
    res = _hyp_linear(xh, w_ref[...], b_ref[...])
    o_ref[...] = _logmap0(res)


def _stage2_body(p0_ref, p1_ref, w_ref, b_ref, o_ref):
    agg = p0_ref[...] + p1_ref[...]
    out = _proj(_expmap0(agg))
    out = _proj(_expmap0(jax.nn.relu(_logmap0(out))))
    res = _hyp_linear(out, w_ref[...], b_ref[...])
    o_ref[...] = _logmap0(res)


def _stage3_body(p0_ref, p1_ref, wd_ref, bd_ref, o_ref):
    agg = p0_ref[...] + p1_ref[...]
    h = _proj(_expmap0(agg))
    ht = _logmap0(h)
    logits = lax.dot_general(ht, wd_ref[...], (((1,), (1,)), ((), ())),
                             preferred_element_type=jnp.float32)
    logits = logits + bd_ref[...]
    m = jnp.max(logits, axis=-1, keepdims=True)
    sh = logits - m
    lse = jnp.log(jnp.sum(jnp.exp(sh), axis=-1, keepdims=True))
    o_ref[...] = sh - lse


def _row_spec():
    return pl.BlockSpec((ROWS_PER_BLOCK, D_HID), lambda i: (i, 0))


def _part_spec(k):
    return pl.BlockSpec((1, ROWS_PER_BLOCK, D_HID), lambda i, k=k: (k, i, 0))


def _full_spec(r, c):
    return pl.BlockSpec((r, c), lambda i: (0, 0))


def _tc_stage1(x, W1, b1):
    return pl.pallas_call(
        _stage1_body,
        grid=(N_BLOCKS,),
        in_specs=[_row_spec(), _full_spec(D_HID, D_FEAT), _full_spec(1, D_HID)],
        out_specs=_row_spec(),
        out_shape=jax.ShapeDtypeStruct((N_NODES, D_HID), jnp.float32),
    )(x, W1, b1.reshape(1, D_HID))


def _tc_stage2(p0, p1, W2, b2):
    return pl.pallas_call(
        _stage2_body,
        grid=(N_BLOCKS,),
        in_specs=[_row_spec(), _row_spec(), _full_spec(D_HID, D_HID),
                  _full_spec(1, D_HID)],
        out_specs=_row_spec(),
        out_shape=jax.ShapeDtypeStruct((N_NODES, D_HID), jnp.float32),
    )(p0, p1, W2, b2.reshape(1, D_HID))


def _tc_stage3(p0, p1, Wd, bd):
    return pl.pallas_call(
        _stage3_body,
        grid=(N_BLOCKS,),
        in_specs=[_row_spec(), _row_spec(), _full_spec(N_CLASSES, D_HID),
                  _full_spec(1, N_CLASSES)],
        out_specs=pl.BlockSpec((ROWS_PER_BLOCK, N_CLASSES), lambda i: (i, 0)),
        out_shape=jax.ShapeDtypeStruct((N_NODES, N_CLASSES), jnp.float32),
    )(p0, p1, Wd, bd.reshape(1, N_CLASSES))


RING = 3


def _sc_agg_kernel(xt_hbm, src_hbm, dst_hbm, w16_hbm, zero_hbm, out_hbm,
                   *sc):
    cid = lax.axis_index("c")
    sid = lax.axis_index("s")
    wid = cid * N_SC_SUBCORES + sid

    sidx = sc[0:3]
    didx = sc[3:5]
    w16s = sc[5:8]
    rows = sc[8:11]
    sbuf = sc[11:13]
    acc = sc[13]
    gsems = sc[14:17]
    wsems = sc[17:20]
    isems = sc[20:23]
    idsems = sc[23:25]
    ssems = sc[25:27]
    LAST = CHUNKS_PER_TILE - 1

    pltpu.sync_copy(zero_hbm, acc.at[pl.ds(sid * ROWS_MAIN, ROWS_MAIN)])

    @pl.when(sid == N_SC_SUBCORES - 1)
    def _zero_rem():
        pltpu.sync_copy(zero_hbm.at[pl.ds(0, ROWS_REM)],
                        acc.at[pl.ds(N_SC_SUBCORES * ROWS_MAIN, ROWS_REM)])

    plsc.subcore_barrier()

    def _fire_sidx(r, c):
        pltpu.async_copy(src_hbm.at[wid, c], sidx[r], isems[r])

    def _wait_sidx(r):
        pltpu.make_async_copy(src_hbm.at[wid, LAST], sidx[r],
                              isems[r]).wait()

    def _fire_gather(r, c):
        pltpu.async_copy(w16_hbm.at[wid, c], w16s[r], wsems[r])
        pltpu.async_copy(xt_hbm.at[sidx[r]], rows[r], gsems[r])

    def _wait_gather(r):
        pltpu.make_async_copy(xt_hbm.at[sidx[r]], rows[r], gsems[r]).wait()
        pltpu.make_async_copy(w16_hbm.at[wid, LAST], w16s[r],
                              wsems[r]).wait()

    def _wait_scatter(b):
        pltpu.make_async_copy(sbuf[b], acc.at[didx[b]], ssems[b]).wait()

    def _scale(r, b):
        for e in range(EDGE_CHUNK):
            wv = w16s[r][e, :]
            for cb in range(8):
                sl = pl.ds(cb * 16, 16)
                sbuf[b][e, sl] = rows[r][e, sl] * wv

    def _chunk_body(c, r, b, wait_scatter):
        _wait_gather(r)
        r2 = (r + 2) % RING
        _wait_sidx(r2)
        _fire_gather(r2, jnp.minimum(c + 2, LAST))
        if wait_scatter is None:
            _wait_scatter(b)
        else:
            @pl.when(wait_scatter)
            def _wait_prev():
                _wait_scatter(b)
        _fire_sidx(r, jnp.minimum(c + 3, LAST))
        pltpu.async_copy(dst_hbm.at[wid, c], didx[b], idsems[b])
        _scale(r, b)
        pltpu.make_async_copy(dst_hbm.at[wid, c], didx[b],
                              idsems[b]).wait()
        pltpu.async_copy(sbuf[b], acc.at[didx[b]], ssems[b], add=True)

    for k in range(RING):
        _fire_sidx(k, k)
    for k in range(RING - 1):
        _wait_sidx(k)
        _fire_gather(k, k)

    def group(i, carry):
        for u in range(RING):
            c = RING * i + u
            guard = (i > 0) if u < 2 else None
            _chunk_body(c, u, u % 2, guard)
        return carry

    n_groups = (CHUNKS_PER_TILE - 4) // RING
    lax.fori_loop(0, n_groups, group, 0)
    g0 = RING * n_groups
    _chunk_body(g0, 0, 0, None)
    _chunk_body(g0 + 1, 1, 1, None)
    _chunk_body(g0 + 2, 2, 0, None)
    _chunk_body(g0 + 3, 0, 1, None)

    _wait_sidx(0)
    _wait_gather(1)
    _wait_gather(2)
    _wait_scatter(0)
    _wait_scatter(1)

    plsc.subcore_barrier()
    r0 = sid * ROWS_MAIN
    pltpu.sync_copy(acc.at[pl.ds(r0, ROWS_MAIN)],
                    out_hbm.at[cid, pl.ds(r0, ROWS_MAIN)])

    @pl.when(sid == N_SC_SUBCORES - 1)
    def _write_rem():
        rr = N_SC_SUBCORES * ROWS_MAIN
        pltpu.sync_copy(acc.at[pl.ds(rr, ROWS_REM)],
                        out_hbm.at[cid, pl.ds(rr, ROWS_REM)])


def _sc_agg(xt, src, dst, w16, zeros_tile):
    mesh = plsc.VectorSubcoreMesh(core_axis_name="c", subcore_axis_name="s")
    f = functools.partial(
        pl.kernel,
        out_type=jax.ShapeDtypeStruct((N_SC_CORES, N_NODES, D_HID),
                                      jnp.float32),
        mesh=mesh,
        scratch_types=(
            [pltpu.VMEM((EDGE_CHUNK,), jnp.int32)] * 5
            + [pltpu.VMEM((EDGE_CHUNK, 16), jnp.float32)] * 3
            + [pltpu.VMEM((EDGE_CHUNK, D_HID), jnp.float32)] * 5
            + [pltpu.VMEM_SHARED((N_NODES, D_HID), jnp.float32)]
            + [pltpu.SemaphoreType.DMA] * 13
        ),
    )(_sc_agg_kernel)
    return f(xt, src, dst, w16, zeros_tile)


def kernel(x, edge_index, edge_weight, W1, b1, W2, b2, Wd, bd):
    n_tiles = N_SC_CORES * N_SC_SUBCORES
    src = edge_index[0].astype(jnp.int32).reshape(
        n_tiles, CHUNKS_PER_TILE, EDGE_CHUNK)
    dst = edge_index[1].astype(jnp.int32).reshape(
        n_tiles, CHUNKS_PER_TILE, EDGE_CHUNK)
    w16 = jnp.broadcast_to(edge_weight.astype(jnp.float32)[:, None],
                           (N_EDGES, 16)).reshape(
        n_tiles, CHUNKS_PER_TILE, EDGE_CHUNK, 16)
    zeros_tile = jnp.zeros((ROWS_MAIN, D_HID), jnp.float32)

    xt1 = _tc_stage1(x, W1, b1)
    parts1 = _sc_agg(xt1, src, dst, w16, zeros_tile)
    xt2 = _tc_stage2(parts1[0], parts1[1], W2, b2)
    parts2 = _sc_agg(xt2, src, dst, w16, zeros_tile)
    return _tc_stage3(parts2[0], parts2[1], Wd, bd)


# speedup vs baseline: 1.0433x; 1.0012x over previous
import functools

import jax
import jax.numpy as jnp
from jax import lax
from jax.experimental import pallas as pl
from jax.experimental.pallas import tpu as pltpu
from jax.experimental.pallas import tpu_sc as plsc

N_NODES = 10000
D_FEAT = 128
D_HID = 128
N_CLASSES = 40
N_EDGES = 320000
MIN_NORM = 1e-15
MAXNORM = 1.0 - 1e-5

ROWS_PER_BLOCK = 1000
N_BLOCKS = N_NODES // ROWS_PER_BLOCK

EDGE_CHUNK = 40
N_SC_CORES = 2
N_SC_SUBCORES = 16
EDGES_PER_TILE = N_EDGES // (N_SC_CORES * N_SC_SUBCORES)
CHUNKS_PER_TILE = EDGES_PER_TILE // EDGE_CHUNK
ROWS_MAIN = 624
ROWS_REM = N_NODES - N_SC_SUBCORES * ROWS_MAIN


def _artanh(z):
    z = jnp.clip(z, -1.0 + 1e-7, 1.0 - 1e-7)
    return 0.5 * jnp.log((1.0 + z) / (1.0 - z))


def _tanh(z):
    return jnp.tanh(jnp.clip(z, -15.0, 15.0))


def _norm(v):
    return jnp.sqrt(jnp.sum(v * v, axis=-1, keepdims=True))


def _proj(v):
    n = jnp.maximum(_norm(v), MIN_NORM)
    return jnp.where(n > MAXNORM, v / n * MAXNORM, v)


def _expmap0(u):
    n = jnp.maximum(_norm(u), MIN_NORM)
    return _tanh(n) * u / n


def _logmap0(p):
    n = jnp.maximum(_norm(p), MIN_NORM)
    return p / n * _artanh(n)


def _mobius_add(a, b):
    a2 = jnp.sum(a * a, axis=-1, keepdims=True)
    b2 = jnp.sum(b * b, axis=-1, keepdims=True)
    ab = jnp.sum(a * b, axis=-1, keepdims=True)
    num = (1.0 + 2.0 * ab + b2) * a + (1.0 - a2) * b
    den = 1.0 + 2.0 * ab + a2 * b2
    return num / jnp.maximum(den, MIN_NORM)


def _hyp_linear(h, W, brow):
    hn = jnp.maximum(_norm(h), MIN_NORM)
    mx = lax.dot_general(h, W, (((1,), (1,)), ((), ())),
                         preferred_element_type=jnp.float32)
    mxn = jnp.maximum(_norm(mx), MIN_NORM)
    mv = _tanh(mxn / hn * _artanh(hn)) * mx / mxn
    mv = _proj(mv)
    bias = _proj(_expmap0(brow))
    return _proj(_mobius_add(mv, bias))


def _stage1_body(x_ref, w_ref, b_ref, o_ref):
    x = x_ref[...]
    xh = _proj(_expmap0(x))
    res = _hyp_linear(xh, w_ref[...], b_ref[...])
    o_ref[...] = _logmap0(res)


def _stage2_body(p0_ref, p1_ref, w_ref, b_ref, o_ref):
    agg = p0_ref[...] + p1_ref[...]
    out = _proj(_expmap0(agg))
    out = _proj(_expmap0(jax.nn.relu(_logmap0(out))))
    res = _hyp_linear(out, w_ref[...], b_ref[...])
    o_ref[...] = _logmap0(res)


def _stage3_body(p0_ref, p1_ref, wd_ref, bd_ref, o_ref):
    agg = p0_ref[...] + p1_ref[...]
    h = _proj(_expmap0(agg))
    ht = _logmap0(h)
    logits = lax.dot_general(ht, wd_ref[...], (((1,), (1,)), ((), ())),
                             preferred_element_type=jnp.float32)
    logits = logits + bd_ref[...]
    m = jnp.max(logits, axis=-1, keepdims=True)
    sh = logits - m
    lse = jnp.log(jnp.sum(jnp.exp(sh), axis=-1, keepdims=True))
    o_ref[...] = sh - lse


def _row_spec():
    return pl.BlockSpec((ROWS_PER_BLOCK, D_HID), lambda i: (i, 0))


def _full_spec(r, c):
    return pl.BlockSpec((r, c), lambda i: (0, 0))


def _tc_stage1(x, W1, b1):
    return pl.pallas_call(
        _stage1_body,
        grid=(N_BLOCKS,),
        in_specs=[_row_spec(), _full_spec(D_HID, D_FEAT), _full_spec(1, D_HID)],
        out_specs=_row_spec(),
        out_shape=jax.ShapeDtypeStruct((N_NODES, D_HID), jnp.float32),
    )(x, W1, b1.reshape(1, D_HID))


def _tc_stage2(p0, p1, W2, b2):
    return pl.pallas_call(
        _stage2_body,
        grid=(N_BLOCKS,),
        in_specs=[_row_spec(), _row_spec(), _full_spec(D_HID, D_HID),
                  _full_spec(1, D_HID)],
        out_specs=_row_spec(),
        out_shape=jax.ShapeDtypeStruct((N_NODES, D_HID), jnp.float32),
    )(p0, p1, W2, b2.reshape(1, D_HID))


def _tc_stage3(p0, p1, Wd, bd):
    return pl.pallas_call(
        _stage3_body,
        grid=(N_BLOCKS,),
        in_specs=[_row_spec(), _row_spec(), _full_spec(N_CLASSES, D_HID),
                  _full_spec(1, N_CLASSES)],
        out_specs=pl.BlockSpec((ROWS_PER_BLOCK, N_CLASSES), lambda i: (i, 0)),
        out_shape=jax.ShapeDtypeStruct((N_NODES, N_CLASSES), jnp.float32),
    )(p0, p1, Wd, bd.reshape(1, N_CLASSES))


RING = 3


def _sc_agg_kernel(xt_hbm, src_hbm, dst_hbm, w16_hbm, zero_hbm, out_hbm,
                   *sc):
    cid = lax.axis_index("c")
    sid = lax.axis_index("s")
    wid = cid * N_SC_SUBCORES + sid

    sidx = sc[0:3]
    didx = sc[3:5]
    w16s = sc[5:8]
    rows = sc[8:11]
    sbuf = sc[11:13]
    acc = sc[13]
    gsems = sc[14:17]
    wsems = sc[17:20]
    isems = sc[20:23]
    idsems = sc[23:25]
    ssems = sc[25:27]
    LAST = CHUNKS_PER_TILE - 1

    pltpu.sync_copy(zero_hbm, acc.at[pl.ds(sid * ROWS_MAIN, ROWS_MAIN)])

    @pl.when(sid == N_SC_SUBCORES - 1)
    def _zero_rem():
        pltpu.sync_copy(zero_hbm.at[pl.ds(0, ROWS_REM)],
                        acc.at[pl.ds(N_SC_SUBCORES * ROWS_MAIN, ROWS_REM)])

    plsc.subcore_barrier()

    def _fire_sidx(r, c):
        pltpu.async_copy(src_hbm.at[wid, c], sidx[r], isems[r])

    def _wait_sidx(r):
        pltpu.make_async_copy(src_hbm.at[wid, LAST], sidx[r],
                              isems[r]).wait()

    def _fire_gather(r, c):
        pltpu.async_copy(w16_hbm.at[wid, c], w16s[r], wsems[r])
        pltpu.async_copy(xt_hbm.at[sidx[r]], rows[r], gsems[r])

    def _wait_gather(r):
        pltpu.make_async_copy(xt_hbm.at[sidx[r]], rows[r], gsems[r]).wait()
        pltpu.make_async_copy(w16_hbm.at[wid, LAST], w16s[r],
                              wsems[r]).wait()

    def _wait_scatter(b):
        pltpu.make_async_copy(sbuf[b], acc.at[didx[b]], ssems[b]).wait()

    def _scale(r, b):
        for e in range(EDGE_CHUNK):
            wv = w16s[r][e, :]
            for cb in range(8):
                sl = pl.ds(cb * 16, 16)
                sbuf[b][e, sl] = rows[r][e, sl] * wv

    def _chunk_body(c, r, b, wait_scatter):
        _wait_gather(r)
        r2 = (r + 2) % RING
        _wait_sidx(r2)
        _fire_gather(r2, jnp.minimum(c + 2, LAST))
        if wait_scatter is None:
            _wait_scatter(b)
        else:
            @pl.when(wait_scatter)
            def _wait_prev():
                _wait_scatter(b)
        _fire_sidx(r, jnp.minimum(c + 3, LAST))
        pltpu.async_copy(dst_hbm.at[wid, c], didx[b], idsems[b])
        _scale(r, b)
        pltpu.make_async_copy(dst_hbm.at[wid, c], didx[b],
                              idsems[b]).wait()
        pltpu.async_copy(sbuf[b], acc.at[didx[b]], ssems[b], add=True)

    for k in range(RING):
        _fire_sidx(k, k)
    for k in range(RING - 1):
        _wait_sidx(k)
        _fire_gather(k, k)

    def group(i, carry):
        for u in range(RING):
            c = RING * i + u
            guard = (i > 0) if u < 2 else None
            _chunk_body(c, u, u % 2, guard)
        return carry

    n_groups = (CHUNKS_PER_TILE - 4) // RING
    lax.fori_loop(0, n_groups, group, 0)
    g0 = RING * n_groups
    _chunk_body(g0, 0, 0, None)
    _chunk_body(g0 + 1, 1, 1, None)
    _chunk_body(g0 + 2, 2, 0, None)
    _chunk_body(g0 + 3, 0, 1, None)

    _wait_sidx(0)
    _wait_gather(1)
    _wait_gather(2)
    _wait_scatter(0)
    _wait_scatter(1)

    plsc.subcore_barrier()
    r0 = sid * ROWS_MAIN
    pltpu.sync_copy(acc.at[pl.ds(r0, ROWS_MAIN)],
                    out_hbm.at[cid, pl.ds(r0, ROWS_MAIN)])

    @pl.when(sid == N_SC_SUBCORES - 1)
    def _write_rem():
        rr = N_SC_SUBCORES * ROWS_MAIN
        pltpu.sync_copy(acc.at[pl.ds(rr, ROWS_REM)],
                        out_hbm.at[cid, pl.ds(rr, ROWS_REM)])


def _sc_agg(xt, src, dst, w16, zeros_tile):
    mesh = plsc.VectorSubcoreMesh(core_axis_name="c", subcore_axis_name="s")
    f = functools.partial(
        pl.kernel,
        out_type=jax.ShapeDtypeStruct((N_SC_CORES, N_NODES, D_HID),
                                      jnp.float32),
        mesh=mesh,
        scratch_types=(
            [pltpu.VMEM((EDGE_CHUNK,), jnp.int32)] * 5
            + [pltpu.VMEM((EDGE_CHUNK, 16), jnp.float32)] * 3
            + [pltpu.VMEM((EDGE_CHUNK, D_HID), jnp.float32)] * 5
            + [pltpu.VMEM_SHARED((N_NODES, D_HID), jnp.float32)]
            + [pltpu.SemaphoreType.DMA] * 13
        ),
    )(_sc_agg_kernel)
    return f(xt, src, dst, w16, zeros_tile)


def kernel(x, edge_index, edge_weight, W1, b1, W2, b2, Wd, bd):
    n_tiles = N_SC_CORES * N_SC_SUBCORES
    src = edge_index[0].astype(jnp.int32).reshape(
        n_tiles, CHUNKS_PER_TILE, EDGE_CHUNK)
    dst = edge_index[1].astype(jnp.int32).reshape(
        n_tiles, CHUNKS_PER_TILE, EDGE_CHUNK)
    w16 = jnp.broadcast_to(edge_weight.astype(jnp.float32)[:, None],
                           (N_EDGES, 16)).reshape(
        n_tiles, CHUNKS_PER_TILE, EDGE_CHUNK, 16)
    zeros_tile = jnp.zeros((ROWS_MAIN, D_HID), jnp.float32)

    xt1 = _tc_stage1(x, W1, b1)
    parts1 = _sc_agg(xt1, src, dst, w16, zeros_tile)
    xt2 = _tc_stage2(parts1[0], parts1[1], W2, b2)
    parts2 = _sc_agg(xt2, src, dst, w16, zeros_tile)
    return _tc_stage3(parts2[0], parts2[1], Wd, bd)
